# Initial kernel scaffold; baseline (speedup 1.0000x reference)
import jax
import jax.numpy as jnp
from jax.experimental import pallas as pl


def kernel(x, edge_index, edge_label_index, W_pre1, b_pre1, W_pre2, b_pre2, W_c1, b_c1, W_c2, b_c2, W_post, b_post):
    raise NotImplementedError("write your pallas kernel here")


# speedup vs baseline: 5.1609x; 5.1609x over previous
import functools

import jax
import jax.numpy as jnp
from jax import lax
from jax.experimental import pallas as pl
from jax.experimental.pallas import tpu as pltpu
from jax.experimental.pallas import tpu_sc as plsc

N = 10000
D = 128
E = 320000
EL = 100000

NC = 2
NS = 16
NW = NC * NS

NPAD = 10240
EPAD = 327680
ELPAD = 102400
ECH = EPAD // NW // 128
SCH = ELPAD // NW // 128
RPT = NPAD // NS


def _mesh():
    return plsc.VectorSubcoreMesh(
        core_axis_name="c", subcore_axis_name="s", num_cores=NC, num_subcores=NS)


def _sc_degree(dst2):
    @functools.partial(
        pl.kernel,
        out_type=jax.ShapeDtypeStruct((NC, NPAD, 16), jnp.float32),
        mesh=_mesh(),
        scratch_types=[
            pltpu.VMEM((ECH, 128), jnp.int32),
            pltpu.VMEM((128, 16), jnp.float32),
            pltpu.VMEM((128, 16), jnp.float32),
            pltpu.VMEM_SHARED((NPAD, 16), jnp.float32),
        ],
    )
    def deg_kernel(dst_hbm, out_hbm, dst_v, ones_v, zero_v, acc):
        cid = lax.axis_index("c")
        sid = lax.axis_index("s")
        wid = sid * NC + cid

        def fill(i, _):
            ones_v[i, :] = jnp.full((16,), 1.0, jnp.float32)
            zero_v[i, :] = jnp.zeros((16,), jnp.float32)
            return 0
        lax.fori_loop(0, 128, fill, 0)
        for k in range(RPT // 128):
            pltpu.sync_copy(zero_v, acc.at[pl.ds(sid * RPT + k * 128, 128)])
        plsc.subcore_barrier()

        pltpu.sync_copy(dst_hbm.at[pl.ds(wid * ECH, ECH)], dst_v)

        def chunk(j, _):
            pltpu.sync_copy(ones_v, acc.at[dst_v.at[j]], add=True)
            return 0
        lax.fori_loop(0, ECH, chunk, 0)
        plsc.subcore_barrier()
        for k in range(RPT // 128):
            r0 = sid * RPT + k * 128
            pltpu.sync_copy(acc.at[pl.ds(r0, 128)], out_hbm.at[cid, pl.ds(r0, 128)])

    return deg_kernel(dst2)


def _sc_conv(g, src2, dst2):
    @functools.partial(
        pl.kernel,
        out_type=jax.ShapeDtypeStruct((NC, NPAD, D), jnp.float32),
        mesh=_mesh(),
        scratch_types=[
            pltpu.VMEM((ECH, 128), jnp.int32),
            pltpu.VMEM((ECH, 128), jnp.int32),
            pltpu.VMEM((128, D), jnp.float32),
            pltpu.SemaphoreType.DMA,
            pltpu.VMEM_SHARED((NPAD, D), jnp.float32),
        ],
    )
    def conv_kernel(g_hbm, src_hbm, dst_hbm, out_hbm, src_v, dst_v, rows, sem, acc):
        cid = lax.axis_index("c")
        sid = lax.axis_index("s")
        wid = sid * NC + cid

        def zrow(i, _):
            for k in range(D // 16):
                rows[i, pl.ds(k * 16, 16)] = jnp.zeros((16,), jnp.float32)
            return 0
        lax.fori_loop(0, 128, zrow, 0)
        for k in range(RPT // 128):
            pltpu.sync_copy(rows, acc.at[pl.ds(sid * RPT + k * 128, 128)])
        plsc.subcore_barrier()

        pltpu.sync_copy(src_hbm.at[pl.ds(wid * ECH, ECH)], src_v)
        pltpu.sync_copy(dst_hbm.at[pl.ds(wid * ECH, ECH)], dst_v)

        def chunk(j, _):
            pltpu.async_copy(g_hbm.at[src_v.at[j]], rows, sem).wait()
            pltpu.sync_copy(rows, acc.at[dst_v.at[j]], add=True)
            return 0
        lax.fori_loop(0, ECH, chunk, 0)
        plsc.subcore_barrier()
        for k in range(RPT // 128):
            r0 = sid * RPT + k * 128
            pltpu.sync_copy(acc.at[pl.ds(r0, 128)], out_hbm.at[cid, pl.ds(r0, 128)])

    return conv_kernel(g, src2, dst2)


def _sc_gather_pairs(t1, t2, els, eld):
    ept = SCH * 128

    @functools.partial(
        pl.kernel,
        out_type=[
            jax.ShapeDtypeStruct((ELPAD, D), jnp.float32),
            jax.ShapeDtypeStruct((ELPAD, D), jnp.float32),
        ],
        mesh=_mesh(),
        scratch_types=[
            pltpu.VMEM((ept,), jnp.int32),
            pltpu.VMEM((ept,), jnp.int32),
            pltpu.VMEM((128, D), jnp.float32),
            pltpu.VMEM((128, D), jnp.float32),
            pltpu.SemaphoreType.DMA,
            pltpu.SemaphoreType.DMA,
        ],
    )
    def gather_kernel(t1_hbm, t2_hbm, els_hbm, eld_hbm, outa_hbm, outb_hbm,
                      els_v, eld_v, rows_a, rows_b, sem_a, sem_b):
        cid = lax.axis_index("c")
        sid = lax.axis_index("s")
        wid = sid * NC + cid
        base = wid * ept

        pltpu.sync_copy(els_hbm.at[pl.ds(base, ept)], els_v)
        pltpu.sync_copy(eld_hbm.at[pl.ds(base, ept)], eld_v)

        def chunk(j, _):
            cp_a = pltpu.async_copy(
                t1_hbm.at[els_v.at[pl.ds(j * 128, 128)]], rows_a, sem_a)
            cp_b = pltpu.async_copy(
                t2_hbm.at[eld_v.at[pl.ds(j * 128, 128)]], rows_b, sem_b)
            cp_a.wait()
            cp_b.wait()
            r0 = base + j * 128
            pltpu.sync_copy(rows_a, outa_hbm.at[pl.ds(r0, 128)])
            pltpu.sync_copy(rows_b, outb_hbm.at[pl.ds(r0, 128)])
            return 0
        lax.fori_loop(0, SCH, chunk, 0)

    return gather_kernel(t1, t2, els, eld)


def _tc_score(pa, pb):
    R = 2048

    def body(a_ref, b_ref, out_ref):
        out_ref[...] = jnp.sum(a_ref[...] * b_ref[...], axis=1, keepdims=True)

    return pl.pallas_call(
        body,
        grid=(ELPAD // R,),
        in_specs=[
            pl.BlockSpec((R, D), lambda i: (i, 0)),
            pl.BlockSpec((R, D), lambda i: (i, 0)),
        ],
        out_specs=pl.BlockSpec((R, 1), lambda i: (i, 0)),
        out_shape=jax.ShapeDtypeStruct((ELPAD, 1), jnp.float32),
    )(pa, pb)


def _dinv_from(degp):
    d = degp[0, :, 0:1] + degp[1, :, 0:1] + 1.0
    return lax.rsqrt(d)


def _tc_pre(x, w1t, b1, w2t, b2, wc1t, degp):
    R = 1024

    def body(x_ref, w1_ref, b1_ref, w2_ref, b2_ref, wc1_ref, degp_ref, g1_ref):
        xv = x_ref[...]
        h = jnp.dot(xv, w1_ref[...], preferred_element_type=jnp.float32) + b1_ref[...]
        h = jnp.maximum(h, 0.01 * h)
        h = jnp.dot(h, w2_ref[...], preferred_element_type=jnp.float32) + b2_ref[...]
        h = jnp.maximum(h, 0.01 * h)
        hw = jnp.dot(h, wc1_ref[...], preferred_element_type=jnp.float32)
        g1_ref[...] = hw * _dinv_from(degp_ref[...])

    return pl.pallas_call(
        body,
        grid=(NPAD // R,),
        in_specs=[
            pl.BlockSpec((R, D), lambda i: (i, 0)),
            pl.BlockSpec((D, 256), lambda i: (0, 0)),
            pl.BlockSpec((1, 256), lambda i: (0, 0)),
            pl.BlockSpec((256, D), lambda i: (0, 0)),
            pl.BlockSpec((1, D), lambda i: (0, 0)),
            pl.BlockSpec((D, D), lambda i: (0, 0)),
            pl.BlockSpec((NC, R, 16), lambda i: (0, i, 0)),
        ],
        out_specs=pl.BlockSpec((R, D), lambda i: (i, 0)),
        out_shape=jax.ShapeDtypeStruct((NPAD, D), jnp.float32),
    )(x, w1t, b1, w2t, b2, wc1t, degp)


def _tc_mid(p, g1, degp, bc, wnt):
    R = 1024

    def body(p_ref, g_ref, degp_ref, bc_ref, w_ref, emb_ref, gn_ref):
        pv = p_ref[...]
        dinv = _dinv_from(degp_ref[...])
        z = (pv[0] + pv[1] + g_ref[...]) * dinv + bc_ref[...]
        emb = jnp.maximum(z, 0.01 * z)
        emb_ref[...] = emb
        gn_ref[...] = jnp.dot(emb, w_ref[...], preferred_element_type=jnp.float32) * dinv

    return pl.pallas_call(
        body,
        grid=(NPAD // R,),
        in_specs=[
            pl.BlockSpec((NC, R, D), lambda i: (0, i, 0)),
            pl.BlockSpec((R, D), lambda i: (i, 0)),
            pl.BlockSpec((NC, R, 16), lambda i: (0, i, 0)),
            pl.BlockSpec((1, D), lambda i: (0, 0)),
            pl.BlockSpec((D, D), lambda i: (0, 0)),
        ],
        out_specs=[
            pl.BlockSpec((R, D), lambda i: (i, 0)),
            pl.BlockSpec((R, D), lambda i: (i, 0)),
        ],
        out_shape=[
            jax.ShapeDtypeStruct((NPAD, D), jnp.float32),
            jax.ShapeDtypeStruct((NPAD, D), jnp.float32),
        ],
    )(p, g1, degp, bc, wnt)


def _tc_post(p, g2, degp, bc, wv):
    R = 1024

    def body(p_ref, g_ref, degp_ref, bc_ref, wv_ref, emb_ref, embw_ref):
        pv = p_ref[...]
        dinv = _dinv_from(degp_ref[...])
        z = (pv[0] + pv[1] + g_ref[...]) * dinv + bc_ref[...]
        emb = jnp.maximum(z, 0.01 * z)
        emb_ref[...] = emb
        embw_ref[...] = emb * wv_ref[...]

    return pl.pallas_call(
        body,
        grid=(NPAD // R,),
        in_specs=[
            pl.BlockSpec((NC, R, D), lambda i: (0, i, 0)),
            pl.BlockSpec((R, D), lambda i: (i, 0)),
            pl.BlockSpec((NC, R, 16), lambda i: (0, i, 0)),
            pl.BlockSpec((1, D), lambda i: (0, 0)),
            pl.BlockSpec((1, D), lambda i: (0, 0)),
        ],
        out_specs=[
            pl.BlockSpec((R, D), lambda i: (i, 0)),
            pl.BlockSpec((R, D), lambda i: (i, 0)),
        ],
        out_shape=[
            jax.ShapeDtypeStruct((NPAD, D), jnp.float32),
            jax.ShapeDtypeStruct((NPAD, D), jnp.float32),
        ],
    )(p, g2, degp, bc, wv)


def kernel(x, edge_index, edge_label_index,
           W_pre1, b_pre1, W_pre2, b_pre2,
           W_c1, b_c1, W_c2, b_c2, W_post, b_post):
    epad = jnp.full((EPAD - E,), N, jnp.int32)
    src2 = jnp.concatenate([edge_index[0], epad]).reshape(EPAD // 128, 128)
    dst2 = jnp.concatenate([edge_index[1], epad]).reshape(EPAD // 128, 128)
    lpad = jnp.full((ELPAD - EL,), N, jnp.int32)
    els2 = jnp.concatenate([edge_label_index[0], lpad])
    eld2 = jnp.concatenate([edge_label_index[1], lpad])
    x_pad = jnp.pad(x, ((0, NPAD - N), (0, 0)))

    degp = _sc_conv(jnp.ones((NPAD, D), jnp.float32), src2, dst2)[:, :, :16]
    g1 = _tc_pre(x_pad, W_pre1.T, b_pre1[None], W_pre2.T, b_pre2[None],
                 W_c1.T, degp)
    p1 = _sc_conv(g1, src2, dst2)
    emb1, g2 = _tc_mid(p1, g1, degp, b_c1[None], W_c2.T)
    p2 = _sc_conv(g2, src2, dst2)
    wv = (W_post[0] + W_post[1])[None]
    emb2, emb2w = _tc_post(p2, g2, degp, b_c2[None], wv)
    pa, pb = _sc_gather_pairs(emb2w, emb2, els2, eld2)
    sco = _tc_score(pa, pb)
    scores = sco.reshape(ELPAD)[:EL] + (b_post[0] + b_post[1])
    return scores, emb1[:N], emb2[:N]


# speedup vs baseline: 7.2786x; 1.4103x over previous
import functools

import jax
import jax.numpy as jnp
from jax import lax
from jax.experimental import pallas as pl
from jax.experimental.pallas import tpu as pltpu
from jax.experimental.pallas import tpu_sc as plsc

N = 10000
D = 128
E = 320000
EL = 100000

NC = 2
NS = 16
NW = NC * NS

NPAD = 10240
EPAD = 327680
ELPAD = 102400
ECH = EPAD // NW // 128
SCH = ELPAD // NW // 128
RPT = NPAD // NS


def _mesh():
    return plsc.VectorSubcoreMesh(
        core_axis_name="c", subcore_axis_name="s", num_cores=NC, num_subcores=NS)


def _sc_conv(g, src2, dst2):
    H = ECH // 2

    @functools.partial(
        pl.kernel,
        out_type=jax.ShapeDtypeStruct((NC, NPAD, D), jnp.float32),
        mesh=_mesh(),
        scratch_types=[
            pltpu.VMEM((H, 128), jnp.int32),
            pltpu.VMEM((H, 128), jnp.int32),
            pltpu.VMEM((128, D), jnp.float32),
            pltpu.VMEM((128, D), jnp.float32),
            pltpu.SemaphoreType.DMA,
            pltpu.SemaphoreType.DMA,
            pltpu.VMEM_SHARED((NPAD, D), jnp.float32),
        ],
    )
    def conv_kernel(g_hbm, src_hbm, dst_hbm, out_hbm,
                    src_v, dst_v, rows0, rows1, gsem, ssem, acc):
        cid = lax.axis_index("c")
        sid = lax.axis_index("s")
        wid = sid * NC + cid
        bufs = (rows0, rows1)

        def start_gather(j, k):
            pltpu.async_copy(g_hbm.at[src_v.at[j]], bufs[k], gsem)

        def wait_gather(j, k):
            pltpu.make_async_copy(g_hbm.at[src_v.at[j]], bufs[k], gsem).wait()

        def start_scatter(j, k):
            pltpu.async_copy(bufs[k], acc.at[dst_v.at[j]], ssem, add=True)

        def wait_scatter(j, k):
            pltpu.make_async_copy(bufs[k], acc.at[dst_v.at[j]], ssem).wait()

        def zrow(i, _):
            for k in range(D // 16):
                rows0[i, pl.ds(k * 16, 16)] = jnp.zeros((16,), jnp.float32)
            return 0
        lax.fori_loop(0, 128, zrow, 0)
        for k in range(RPT // 128):
            pltpu.sync_copy(rows0, acc.at[pl.ds(sid * RPT + k * 128, 128)])
        plsc.subcore_barrier()

        for h in range(2):
            base = wid * ECH + h * H
            pltpu.sync_copy(src_hbm.at[pl.ds(base, H)], src_v)
            pltpu.sync_copy(dst_hbm.at[pl.ds(base, H)], dst_v)

            start_gather(0, 0)
            start_gather(1, 1)
            wait_gather(0, 0)
            start_scatter(0, 0)

            def steps(jj, _):
                for k in range(2):
                    j = 2 * jj + 1 + k
                    wait_scatter(j - 1, k)
                    start_gather(j + 1, k)
                    wait_gather(j, (k + 1) % 2)
                    start_scatter(j, (k + 1) % 2)
                return 0
            lax.fori_loop(0, (H - 2) // 2, steps, 0)

            wait_scatter(H - 2, H % 2)
            wait_gather(H - 1, (H - 1) % 2)
            start_scatter(H - 1, (H - 1) % 2)
            wait_scatter(H - 1, (H - 1) % 2)

        plsc.subcore_barrier()
        for k in range(RPT // 128):
            r0 = sid * RPT + k * 128
            pltpu.sync_copy(acc.at[pl.ds(r0, 128)], out_hbm.at[cid, pl.ds(r0, 128)])

    return conv_kernel(g, src2, dst2)


def _sc_degree128(dst2):
    @functools.partial(
        pl.kernel,
        out_type=jax.ShapeDtypeStruct((NC, NPAD, D), jnp.float32),
        mesh=_mesh(),
        scratch_types=[
            pltpu.VMEM((ECH, 128), jnp.int32),
            pltpu.VMEM((128, D), jnp.float32),
            pltpu.SemaphoreType.DMA,
            pltpu.VMEM_SHARED((NPAD, D), jnp.float32),
        ],
    )
    def deg_kernel(dst_hbm, out_hbm, dst_v, ones_v, ssem, acc):
        cid = lax.axis_index("c")
        sid = lax.axis_index("s")
        wid = sid * NC + cid

        def fill(i, _):
            for k in range(D // 16):
                ones_v[i, pl.ds(k * 16, 16)] = jnp.zeros((16,), jnp.float32)
            return 0
        lax.fori_loop(0, 128, fill, 0)
        for k in range(RPT // 128):
            pltpu.sync_copy(ones_v, acc.at[pl.ds(sid * RPT + k * 128, 128)])

        def fill1(i, _):
            for k in range(D // 16):
                ones_v[i, pl.ds(k * 16, 16)] = jnp.full((16,), 1.0, jnp.float32)
            return 0
        lax.fori_loop(0, 128, fill1, 0)
        plsc.subcore_barrier()

        pltpu.sync_copy(dst_hbm.at[pl.ds(wid * ECH, ECH)], dst_v)

        def start_scatter(j):
            pltpu.async_copy(ones_v, acc.at[dst_v.at[j]], ssem, add=True)

        def wait_scatter(j):
            pltpu.make_async_copy(ones_v, acc.at[dst_v.at[j]], ssem).wait()

        def steps(jj, _):
            for k in range(4):
                j = 4 * jj + k
                start_scatter(j)
            for k in range(4):
                j = 4 * jj + k
                wait_scatter(j)
            return 0
        lax.fori_loop(0, ECH // 4, steps, 0)

        plsc.subcore_barrier()
        for k in range(RPT // 128):
            r0 = sid * RPT + k * 128
            pltpu.sync_copy(acc.at[pl.ds(r0, 128)], out_hbm.at[cid, pl.ds(r0, 128)])

    return deg_kernel(dst2)


def _sc_gather_pairs(t1, t2, els, eld):
    ept = SCH * 128

    @functools.partial(
        pl.kernel,
        out_type=[
            jax.ShapeDtypeStruct((ELPAD, D), jnp.float32),
            jax.ShapeDtypeStruct((ELPAD, D), jnp.float32),
        ],
        mesh=_mesh(),
        scratch_types=[
            pltpu.VMEM((ept,), jnp.int32),
            pltpu.VMEM((ept,), jnp.int32),
            pltpu.VMEM((128, D), jnp.float32),
            pltpu.VMEM((128, D), jnp.float32),
            pltpu.VMEM((128, D), jnp.float32),
            pltpu.VMEM((128, D), jnp.float32),
            pltpu.SemaphoreType.DMA,
            pltpu.SemaphoreType.DMA,
        ],
    )
    def gather_kernel(t1_hbm, t2_hbm, els_hbm, eld_hbm, outa_hbm, outb_hbm,
                      els_v, eld_v, ra0, ra1, rb0, rb1, sem_a, sem_b):
        rows_a = (ra0, ra1)
        rows_b = (rb0, rb1)
        cid = lax.axis_index("c")
        sid = lax.axis_index("s")
        wid = sid * NC + cid
        base = wid * ept

        pltpu.sync_copy(els_hbm.at[pl.ds(base, ept)], els_v)
        pltpu.sync_copy(eld_hbm.at[pl.ds(base, ept)], eld_v)

        def sg(t_hbm, idx_v, rows, j, k):
            pltpu.async_copy(
                t_hbm.at[idx_v.at[pl.ds(j * 128, 128)]], rows[k], sem_a)

        def wg(t_hbm, idx_v, rows, j, k):
            pltpu.make_async_copy(
                t_hbm.at[idx_v.at[pl.ds(j * 128, 128)]], rows[k], sem_a).wait()

        def sw(rows, out_hbm, j, k):
            pltpu.async_copy(rows[k], out_hbm.at[pl.ds(base + j * 128, 128)],
                             sem_b)

        def ww(rows, out_hbm, j, k):
            pltpu.make_async_copy(rows[k],
                                  out_hbm.at[pl.ds(base + j * 128, 128)],
                                  sem_b).wait()

        def step(j, k, first, last):
            for idx_v, rows, t_hbm, out_hbm in (
                (els_v, rows_a, t1_hbm, outa_hbm),
                (eld_v, rows_b, t2_hbm, outb_hbm),
            ):
                if not first:
                    ww(rows, out_hbm, j - 1, k)
                if not last:
                    sg(t_hbm, idx_v, rows, j + 1, k)
                wg(t_hbm, idx_v, rows, j, (k + 1) % 2)
                sw(rows, out_hbm, j, (k + 1) % 2)

        sg(t1_hbm, els_v, rows_a, 0, 0)
        sg(t2_hbm, eld_v, rows_b, 0, 0)
        step(0, 1, True, False)

        def steps(jj, _):
            for k in range(2):
                step(2 * jj + 1 + k, k, False, False)
            return 0
        lax.fori_loop(0, (SCH - 3) // 2, steps, 0)

        step(SCH - 2, 0, False, False)
        step(SCH - 1, 1, False, True)
        ww(rows_a, outa_hbm, SCH - 1, 0)
        ww(rows_b, outb_hbm, SCH - 1, 0)

    return gather_kernel(t1, t2, els, eld)


def _tc_score(pa, pb):
    R = 2048

    def body(a_ref, b_ref, out_ref):
        out_ref[...] = jnp.sum(a_ref[...] * b_ref[...], axis=1, keepdims=True)

    return pl.pallas_call(
        body,
        grid=(ELPAD // R,),
        in_specs=[
            pl.BlockSpec((R, D), lambda i: (i, 0)),
            pl.BlockSpec((R, D), lambda i: (i, 0)),
        ],
        out_specs=pl.BlockSpec((R, 1), lambda i: (i, 0)),
        out_shape=jax.ShapeDtypeStruct((ELPAD, 1), jnp.float32),
    )(pa, pb)


def _dinv_from(degp):
    d = degp[0, :, 0:1] + degp[1, :, 0:1] + 1.0
    return lax.rsqrt(d)


def _tc_pre(x, w1t, b1, w2t, b2, wc1t, degp):
    R = 1024

    def body(x_ref, w1_ref, b1_ref, w2_ref, b2_ref, wc1_ref, degp_ref, g1_ref):
        xv = x_ref[...]
        h = jnp.dot(xv, w1_ref[...], preferred_element_type=jnp.float32) + b1_ref[...]
        h = jnp.maximum(h, 0.01 * h)
        h = jnp.dot(h, w2_ref[...], preferred_element_type=jnp.float32) + b2_ref[...]
        h = jnp.maximum(h, 0.01 * h)
        hw = jnp.dot(h, wc1_ref[...], preferred_element_type=jnp.float32)
        g1_ref[...] = hw * _dinv_from(degp_ref[...])

    return pl.pallas_call(
        body,
        grid=(NPAD // R,),
        in_specs=[
            pl.BlockSpec((R, D), lambda i: (i, 0)),
            pl.BlockSpec((D, 256), lambda i: (0, 0)),
            pl.BlockSpec((1, 256), lambda i: (0, 0)),
            pl.BlockSpec((256, D), lambda i: (0, 0)),
            pl.BlockSpec((1, D), lambda i: (0, 0)),
            pl.BlockSpec((D, D), lambda i: (0, 0)),
            pl.BlockSpec((NC, R, 16), lambda i: (0, i, 0)),
        ],
        out_specs=pl.BlockSpec((R, D), lambda i: (i, 0)),
        out_shape=jax.ShapeDtypeStruct((NPAD, D), jnp.float32),
    )(x, w1t, b1, w2t, b2, wc1t, degp)


def _tc_mid(p, g1, degp, bc, wnt):
    R = 1024

    def body(p_ref, g_ref, degp_ref, bc_ref, w_ref, emb_ref, gn_ref):
        pv = p_ref[...]
        dinv = _dinv_from(degp_ref[...])
        z = (pv[0] + pv[1] + g_ref[...]) * dinv + bc_ref[...]
        emb = jnp.maximum(z, 0.01 * z)
        emb_ref[...] = emb
        gn_ref[...] = jnp.dot(emb, w_ref[...], preferred_element_type=jnp.float32) * dinv

    return pl.pallas_call(
        body,
        grid=(NPAD // R,),
        in_specs=[
            pl.BlockSpec((NC, R, D), lambda i: (0, i, 0)),
            pl.BlockSpec((R, D), lambda i: (i, 0)),
            pl.BlockSpec((NC, R, 16), lambda i: (0, i, 0)),
            pl.BlockSpec((1, D), lambda i: (0, 0)),
            pl.BlockSpec((D, D), lambda i: (0, 0)),
        ],
        out_specs=[
            pl.BlockSpec((R, D), lambda i: (i, 0)),
            pl.BlockSpec((R, D), lambda i: (i, 0)),
        ],
        out_shape=[
            jax.ShapeDtypeStruct((NPAD, D), jnp.float32),
            jax.ShapeDtypeStruct((NPAD, D), jnp.float32),
        ],
    )(p, g1, degp, bc, wnt)


def _tc_post(p, g2, degp, bc, wv):
    R = 1024

    def body(p_ref, g_ref, degp_ref, bc_ref, wv_ref, emb_ref, embw_ref):
        pv = p_ref[...]
        dinv = _dinv_from(degp_ref[...])
        z = (pv[0] + pv[1] + g_ref[...]) * dinv + bc_ref[...]
        emb = jnp.maximum(z, 0.01 * z)
        emb_ref[...] = emb
        embw_ref[...] = emb * wv_ref[...]

    return pl.pallas_call(
        body,
        grid=(NPAD // R,),
        in_specs=[
            pl.BlockSpec((NC, R, D), lambda i: (0, i, 0)),
            pl.BlockSpec((R, D), lambda i: (i, 0)),
            pl.BlockSpec((NC, R, 16), lambda i: (0, i, 0)),
            pl.BlockSpec((1, D), lambda i: (0, 0)),
            pl.BlockSpec((1, D), lambda i: (0, 0)),
        ],
        out_specs=[
            pl.BlockSpec((R, D), lambda i: (i, 0)),
            pl.BlockSpec((R, D), lambda i: (i, 0)),
        ],
        out_shape=[
            jax.ShapeDtypeStruct((NPAD, D), jnp.float32),
            jax.ShapeDtypeStruct((NPAD, D), jnp.float32),
        ],
    )(p, g2, degp, bc, wv)


def kernel(x, edge_index, edge_label_index,
           W_pre1, b_pre1, W_pre2, b_pre2,
           W_c1, b_c1, W_c2, b_c2, W_post, b_post):
    epad = jnp.full((EPAD - E,), N, jnp.int32)
    src2 = jnp.concatenate([edge_index[0], epad]).reshape(EPAD // 128, 128)
    dst2 = jnp.concatenate([edge_index[1], epad]).reshape(EPAD // 128, 128)
    lpad = jnp.full((ELPAD - EL,), N, jnp.int32)
    els2 = jnp.concatenate([edge_label_index[0], lpad])
    eld2 = jnp.concatenate([edge_label_index[1], lpad])
    x_pad = jnp.pad(x, ((0, NPAD - N), (0, 0)))

    degp = _sc_degree128(dst2)[:, :, :16]
    g1 = _tc_pre(x_pad, W_pre1.T, b_pre1[None], W_pre2.T, b_pre2[None],
                 W_c1.T, degp)
    p1 = _sc_conv(g1, src2, dst2)
    emb1, g2 = _tc_mid(p1, g1, degp, b_c1[None], W_c2.T)
    p2 = _sc_conv(g2, src2, dst2)
    wv = (W_post[0] + W_post[1])[None]
    emb2, emb2w = _tc_post(p2, g2, degp, b_c2[None], wv)
    pa, pb = _sc_gather_pairs(emb2w, emb2, els2, eld2)
    sco = _tc_score(pa, pb)
    scores = sco.reshape(ELPAD)[:EL] + (b_post[0] + b_post[1])
    return scores, emb1[:N], emb2[:N]


# speedup vs baseline: 16.2997x; 2.2394x over previous
import functools

import jax
import jax.numpy as jnp
from jax import lax
from jax.experimental import pallas as pl
from jax.experimental.pallas import tpu as pltpu
from jax.experimental.pallas import tpu_sc as plsc

N = 10000
D = 128
E = 320000
EL = 100000

NC = 2
NS = 16
NW = NC * NS

NPAD = 10240
EPAD = 327680
ELPAD = 102400
ECH = EPAD // NW // 128
SCH = ELPAD // NW // 128
RPT = NPAD // NS


def _mesh():
    return plsc.VectorSubcoreMesh(
        core_axis_name="c", subcore_axis_name="s", num_cores=NC, num_subcores=NS)


def _sc_conv(g, src2, dst2):
    H = ECH // 2

    @functools.partial(
        pl.kernel,
        out_type=jax.ShapeDtypeStruct((NC, NPAD, D), jnp.float32),
        mesh=_mesh(),
        scratch_types=[
            pltpu.VMEM((H, 128), jnp.int32),
            pltpu.VMEM((H, 128), jnp.int32),
            pltpu.VMEM((128, D), jnp.float32),
            pltpu.VMEM((128, D), jnp.float32),
            pltpu.SemaphoreType.DMA,
            pltpu.SemaphoreType.DMA,
            pltpu.VMEM_SHARED((NPAD, D), jnp.float32),
        ],
    )
    def conv_kernel(g_hbm, src_hbm, dst_hbm, out_hbm,
                    src_v, dst_v, rows0, rows1, gsem, ssem, acc):
        cid = lax.axis_index("c")
        sid = lax.axis_index("s")
        wid = sid * NC + cid
        bufs = (rows0, rows1)

        def start_gather(j, k):
            pltpu.async_copy(g_hbm.at[src_v.at[j]], bufs[k], gsem)

        def wait_gather(j, k):
            pltpu.make_async_copy(g_hbm.at[src_v.at[j]], bufs[k], gsem).wait()

        def start_scatter(j, k):
            pltpu.async_copy(bufs[k], acc.at[dst_v.at[j]], ssem, add=True)

        def wait_scatter(j, k):
            pltpu.make_async_copy(bufs[k], acc.at[dst_v.at[j]], ssem).wait()

        def zrow(i, _):
            for k in range(D // 16):
                rows0[i, pl.ds(k * 16, 16)] = jnp.zeros((16,), jnp.float32)
            return 0
        lax.fori_loop(0, 128, zrow, 0)
        for k in range(RPT // 128):
            pltpu.sync_copy(rows0, acc.at[pl.ds(sid * RPT + k * 128, 128)])
        plsc.subcore_barrier()

        for h in range(2):
            base = wid * ECH + h * H
            pltpu.sync_copy(src_hbm.at[pl.ds(base, H)], src_v)
            pltpu.sync_copy(dst_hbm.at[pl.ds(base, H)], dst_v)

            start_gather(0, 0)
            start_gather(1, 1)
            wait_gather(0, 0)
            start_scatter(0, 0)

            def steps(jj, _):
                for k in range(2):
                    j = 2 * jj + 1 + k
                    wait_scatter(j - 1, k)
                    start_gather(j + 1, k)
                    wait_gather(j, (k + 1) % 2)
                    start_scatter(j, (k + 1) % 2)
                return 0
            lax.fori_loop(0, (H - 2) // 2, steps, 0)

            wait_scatter(H - 2, H % 2)
            wait_gather(H - 1, (H - 1) % 2)
            start_scatter(H - 1, (H - 1) % 2)
            wait_scatter(H - 1, (H - 1) % 2)

        plsc.subcore_barrier()
        for k in range(RPT // 128):
            r0 = sid * RPT + k * 128
            pltpu.sync_copy(acc.at[pl.ds(r0, 128)], out_hbm.at[cid, pl.ds(r0, 128)])

    return conv_kernel(g, src2, dst2)


def _sc_degree128(dst2):
    @functools.partial(
        pl.kernel,
        out_type=jax.ShapeDtypeStruct((NC, NPAD, D), jnp.float32),
        mesh=_mesh(),
        scratch_types=[
            pltpu.VMEM((ECH, 128), jnp.int32),
            pltpu.VMEM((128, D), jnp.float32),
            pltpu.SemaphoreType.DMA,
            pltpu.VMEM_SHARED((NPAD, D), jnp.float32),
        ],
    )
    def deg_kernel(dst_hbm, out_hbm, dst_v, ones_v, ssem, acc):
        cid = lax.axis_index("c")
        sid = lax.axis_index("s")
        wid = sid * NC + cid

        def fill(i, _):
            for k in range(D // 16):
                ones_v[i, pl.ds(k * 16, 16)] = jnp.zeros((16,), jnp.float32)
            return 0
        lax.fori_loop(0, 128, fill, 0)
        for k in range(RPT // 128):
            pltpu.sync_copy(ones_v, acc.at[pl.ds(sid * RPT + k * 128, 128)])

        def fill1(i, _):
            for k in range(D // 16):
                ones_v[i, pl.ds(k * 16, 16)] = jnp.full((16,), 1.0, jnp.float32)
            return 0
        lax.fori_loop(0, 128, fill1, 0)
        plsc.subcore_barrier()

        pltpu.sync_copy(dst_hbm.at[pl.ds(wid * ECH, ECH)], dst_v)

        def start_scatter(j):
            pltpu.async_copy(ones_v, acc.at[dst_v.at[j]], ssem, add=True)

        def wait_scatter(j):
            pltpu.make_async_copy(ones_v, acc.at[dst_v.at[j]], ssem).wait()

        def steps(jj, _):
            for k in range(4):
                j = 4 * jj + k
                start_scatter(j)
            for k in range(4):
                j = 4 * jj + k
                wait_scatter(j)
            return 0
        lax.fori_loop(0, ECH // 4, steps, 0)

        plsc.subcore_barrier()
        for k in range(RPT // 128):
            r0 = sid * RPT + k * 128
            pltpu.sync_copy(acc.at[pl.ds(r0, 128)], out_hbm.at[cid, pl.ds(r0, 128)])

    return deg_kernel(dst2)


def _sc_gather_pairs(t1, t2, els, eld):
    ept = SCH * 128

    @functools.partial(
        pl.kernel,
        out_type=[
            jax.ShapeDtypeStruct((ELPAD, D), jnp.float32),
            jax.ShapeDtypeStruct((ELPAD, D), jnp.float32),
        ],
        mesh=_mesh(),
        scratch_types=[
            pltpu.VMEM((ept,), jnp.int32),
            pltpu.VMEM((ept,), jnp.int32),
            pltpu.VMEM((128, D), jnp.float32),
            pltpu.VMEM((128, D), jnp.float32),
            pltpu.VMEM((128, D), jnp.float32),
            pltpu.VMEM((128, D), jnp.float32),
            pltpu.SemaphoreType.DMA,
            pltpu.SemaphoreType.DMA,
        ],
    )
    def gather_kernel(t1_hbm, t2_hbm, els_hbm, eld_hbm, outa_hbm, outb_hbm,
                      els_v, eld_v, ra0, ra1, rb0, rb1, sem_a, sem_b):
        rows_a = (ra0, ra1)
        rows_b = (rb0, rb1)
        cid = lax.axis_index("c")
        sid = lax.axis_index("s")
        wid = sid * NC + cid
        base = wid * ept

        pltpu.sync_copy(els_hbm.at[pl.ds(base, ept)], els_v)
        pltpu.sync_copy(eld_hbm.at[pl.ds(base, ept)], eld_v)

        def sg(t_hbm, idx_v, rows, j, k):
            pltpu.async_copy(
                t_hbm.at[idx_v.at[pl.ds(j * 128, 128)]], rows[k], sem_a)

        def wg(t_hbm, idx_v, rows, j, k):
            pltpu.make_async_copy(
                t_hbm.at[idx_v.at[pl.ds(j * 128, 128)]], rows[k], sem_a).wait()

        def sw(rows, out_hbm, j, k):
            pltpu.async_copy(rows[k], out_hbm.at[pl.ds(base + j * 128, 128)],
                             sem_b)

        def ww(rows, out_hbm, j, k):
            pltpu.make_async_copy(rows[k],
                                  out_hbm.at[pl.ds(base + j * 128, 128)],
                                  sem_b).wait()

        def step(j, k, first, last):
            for idx_v, rows, t_hbm, out_hbm in (
                (els_v, rows_a, t1_hbm, outa_hbm),
                (eld_v, rows_b, t2_hbm, outb_hbm),
            ):
                if not first:
                    ww(rows, out_hbm, j - 1, k)
                if not last:
                    sg(t_hbm, idx_v, rows, j + 1, k)
                wg(t_hbm, idx_v, rows, j, (k + 1) % 2)
                sw(rows, out_hbm, j, (k + 1) % 2)

        sg(t1_hbm, els_v, rows_a, 0, 0)
        sg(t2_hbm, eld_v, rows_b, 0, 0)
        step(0, 1, True, False)

        def steps(jj, _):
            for k in range(2):
                step(2 * jj + 1 + k, k, False, False)
            return 0
        lax.fori_loop(0, (SCH - 3) // 2, steps, 0)

        step(SCH - 2, 0, False, False)
        step(SCH - 1, 1, False, True)
        ww(rows_a, outa_hbm, SCH - 1, 0)
        ww(rows_b, outb_hbm, SCH - 1, 0)

    return gather_kernel(t1, t2, els, eld)


def _tc_score(pa, pb):
    R = 2048

    def body(a_ref, b_ref, out_ref):
        out_ref[...] = jnp.sum(a_ref[...] * b_ref[...], axis=1, keepdims=True)

    return pl.pallas_call(
        body,
        grid=(ELPAD // R,),
        in_specs=[
            pl.BlockSpec((R, D), lambda i: (i, 0)),
            pl.BlockSpec((R, D), lambda i: (i, 0)),
        ],
        out_specs=pl.BlockSpec((R, 1), lambda i: (i, 0)),
        out_shape=jax.ShapeDtypeStruct((ELPAD, 1), jnp.float32),
    )(pa, pb)


def _dinv_from(degp):
    d = degp[0, :, 0:1] + degp[1, :, 0:1] + 1.0
    return lax.rsqrt(d)


def _tc_pre(x, w1t, b1, w2t, b2, wc1t, degp):
    R = 1024

    def body(x_ref, w1_ref, b1_ref, w2_ref, b2_ref, wc1_ref, degp_ref, g1_ref):
        xv = x_ref[...]
        h = jnp.dot(xv, w1_ref[...], preferred_element_type=jnp.float32) + b1_ref[...]
        h = jnp.maximum(h, 0.01 * h)
        h = jnp.dot(h, w2_ref[...], preferred_element_type=jnp.float32) + b2_ref[...]
        h = jnp.maximum(h, 0.01 * h)
        hw = jnp.dot(h, wc1_ref[...], preferred_element_type=jnp.float32)
        g1_ref[...] = hw * _dinv_from(degp_ref[...])

    return pl.pallas_call(
        body,
        grid=(NPAD // R,),
        in_specs=[
            pl.BlockSpec((R, D), lambda i: (i, 0)),
            pl.BlockSpec((D, 256), lambda i: (0, 0)),
            pl.BlockSpec((1, 256), lambda i: (0, 0)),
            pl.BlockSpec((256, D), lambda i: (0, 0)),
            pl.BlockSpec((1, D), lambda i: (0, 0)),
            pl.BlockSpec((D, D), lambda i: (0, 0)),
            pl.BlockSpec((NC, R, 16), lambda i: (0, i, 0)),
        ],
        out_specs=pl.BlockSpec((R, D), lambda i: (i, 0)),
        out_shape=jax.ShapeDtypeStruct((NPAD, D), jnp.float32),
    )(x, w1t, b1, w2t, b2, wc1t, degp)


def _tc_mid(p, g1, degp, bc, wnt):
    R = 1024

    def body(p_ref, g_ref, degp_ref, bc_ref, w_ref, emb_ref, gn_ref):
        pv = p_ref[...]
        dinv = _dinv_from(degp_ref[...])
        z = (pv[0] + pv[1] + g_ref[...]) * dinv + bc_ref[...]
        emb = jnp.maximum(z, 0.01 * z)
        emb_ref[...] = emb
        gn_ref[...] = jnp.dot(emb, w_ref[...], preferred_element_type=jnp.float32) * dinv

    return pl.pallas_call(
        body,
        grid=(NPAD // R,),
        in_specs=[
            pl.BlockSpec((NC, R, D), lambda i: (0, i, 0)),
            pl.BlockSpec((R, D), lambda i: (i, 0)),
            pl.BlockSpec((NC, R, 16), lambda i: (0, i, 0)),
            pl.BlockSpec((1, D), lambda i: (0, 0)),
            pl.BlockSpec((D, D), lambda i: (0, 0)),
        ],
        out_specs=[
            pl.BlockSpec((R, D), lambda i: (i, 0)),
            pl.BlockSpec((R, D), lambda i: (i, 0)),
        ],
        out_shape=[
            jax.ShapeDtypeStruct((NPAD, D), jnp.float32),
            jax.ShapeDtypeStruct((NPAD, D), jnp.float32),
        ],
    )(p, g1, degp, bc, wnt)


def _tc_post(p, g2, degp, bc, wv):
    R = 1024

    def body(p_ref, g_ref, degp_ref, bc_ref, wv_ref, emb_ref, embw_ref):
        pv = p_ref[...]
        dinv = _dinv_from(degp_ref[...])
        z = (pv[0] + pv[1] + g_ref[...]) * dinv + bc_ref[...]
        emb = jnp.maximum(z, 0.01 * z)
        emb_ref[...] = emb
        embw_ref[...] = emb * wv_ref[...]

    return pl.pallas_call(
        body,
        grid=(NPAD // R,),
        in_specs=[
            pl.BlockSpec((NC, R, D), lambda i: (0, i, 0)),
            pl.BlockSpec((R, D), lambda i: (i, 0)),
            pl.BlockSpec((NC, R, 16), lambda i: (0, i, 0)),
            pl.BlockSpec((1, D), lambda i: (0, 0)),
            pl.BlockSpec((1, D), lambda i: (0, 0)),
        ],
        out_specs=[
            pl.BlockSpec((R, D), lambda i: (i, 0)),
            pl.BlockSpec((R, D), lambda i: (i, 0)),
        ],
        out_shape=[
            jax.ShapeDtypeStruct((NPAD, D), jnp.float32),
            jax.ShapeDtypeStruct((NPAD, D), jnp.float32),
        ],
    )(p, g2, degp, bc, wv)


def kernel(x, edge_index, edge_label_index,
           W_pre1, b_pre1, W_pre2, b_pre2,
           W_c1, b_c1, W_c2, b_c2, W_post, b_post):
    epad = N + jnp.arange(EPAD - E, dtype=jnp.int32) % (NPAD - N)
    src2 = jnp.concatenate([edge_index[0], epad]).reshape(EPAD // 128, 128)
    dst2 = jnp.concatenate([edge_index[1], epad]).reshape(EPAD // 128, 128)
    lpad = jnp.full((ELPAD - EL,), N, jnp.int32)
    els2 = jnp.concatenate([edge_label_index[0], lpad])
    eld2 = jnp.concatenate([edge_label_index[1], lpad])
    x_pad = jnp.pad(x, ((0, NPAD - N), (0, 0)))

    degp = _sc_degree128(dst2)[:, :, :16]
    g1 = _tc_pre(x_pad, W_pre1.T, b_pre1[None], W_pre2.T, b_pre2[None],
                 W_c1.T, degp)
    p1 = _sc_conv(g1, src2, dst2)
    emb1, g2 = _tc_mid(p1, g1, degp, b_c1[None], W_c2.T)
    p2 = _sc_conv(g2, src2, dst2)
    wv = (W_post[0] + W_post[1])[None]
    emb2, emb2w = _tc_post(p2, g2, degp, b_c2[None], wv)
    pa, pb = _sc_gather_pairs(emb2w, emb2, els2, eld2)
    sco = _tc_score(pa, pb)
    scores = sco.reshape(ELPAD)[:EL] + (b_post[0] + b_post[1])
    return scores, emb1[:N], emb2[:N]


# speedup vs baseline: 19.8244x; 1.2162x over previous
import functools

import jax
import jax.numpy as jnp
from jax import lax
from jax.experimental import pallas as pl
from jax.experimental.pallas import tpu as pltpu
from jax.experimental.pallas import tpu_sc as plsc

N = 10000
D = 128
E = 320000
EL = 100000

NC = 2
NS = 16
NW = NC * NS

NPAD = 10240
EPAD = 327680
ELPAD = 102400
ECH = EPAD // NW // 128
SCH = ELPAD // NW // 128
RPT = NPAD // NS


def _mesh():
    return plsc.VectorSubcoreMesh(
        core_axis_name="c", subcore_axis_name="s", num_cores=NC, num_subcores=NS)


def _sc_conv(g, src2, dst2):
    H = ECH // 2

    @functools.partial(
        pl.kernel,
        out_type=jax.ShapeDtypeStruct((NC, NPAD, D), jnp.float32),
        mesh=_mesh(),
        scratch_types=[
            pltpu.VMEM((H, 128), jnp.int32),
            pltpu.VMEM((H, 128), jnp.int32),
            pltpu.VMEM((128, D), jnp.float32),
            pltpu.VMEM((128, D), jnp.float32),
            pltpu.SemaphoreType.DMA,
            pltpu.SemaphoreType.DMA,
            pltpu.VMEM_SHARED((NPAD, D), jnp.float32),
        ],
    )
    def conv_kernel(g_hbm, src_hbm, dst_hbm, out_hbm,
                    src_v, dst_v, rows0, rows1, gsem, ssem, acc):
        cid = lax.axis_index("c")
        sid = lax.axis_index("s")
        wid = sid * NC + cid
        bufs = (rows0, rows1)

        def start_gather(j, k):
            pltpu.async_copy(g_hbm.at[src_v.at[j]], bufs[k], gsem)

        def wait_gather(j, k):
            pltpu.make_async_copy(g_hbm.at[src_v.at[j]], bufs[k], gsem).wait()

        def start_scatter(j, k):
            pltpu.async_copy(bufs[k], acc.at[dst_v.at[j]], ssem, add=True)

        def wait_scatter(j, k):
            pltpu.make_async_copy(bufs[k], acc.at[dst_v.at[j]], ssem).wait()

        def zrow(i, _):
            for k in range(D // 16):
                rows0[i, pl.ds(k * 16, 16)] = jnp.zeros((16,), jnp.float32)
            return 0
        lax.fori_loop(0, 128, zrow, 0)
        for k in range(RPT // 128):
            pltpu.sync_copy(rows0, acc.at[pl.ds(sid * RPT + k * 128, 128)])
        plsc.subcore_barrier()

        for h in range(2):
            base = wid * ECH + h * H
            pltpu.sync_copy(src_hbm.at[pl.ds(base, H)], src_v)
            pltpu.sync_copy(dst_hbm.at[pl.ds(base, H)], dst_v)

            start_gather(0, 0)
            start_gather(1, 1)
            wait_gather(0, 0)
            start_scatter(0, 0)

            def steps(jj, _):
                for k in range(2):
                    j = 2 * jj + 1 + k
                    wait_scatter(j - 1, k)
                    start_gather(j + 1, k)
                    wait_gather(j, (k + 1) % 2)
                    start_scatter(j, (k + 1) % 2)
                return 0
            lax.fori_loop(0, (H - 2) // 2, steps, 0)

            wait_scatter(H - 2, H % 2)
            wait_gather(H - 1, (H - 1) % 2)
            start_scatter(H - 1, (H - 1) % 2)
            wait_scatter(H - 1, (H - 1) % 2)

        plsc.subcore_barrier()
        for k in range(RPT // 128):
            r0 = sid * RPT + k * 128
            pltpu.sync_copy(acc.at[pl.ds(r0, 128)], out_hbm.at[cid, pl.ds(r0, 128)])

    return conv_kernel(g, src2, dst2)


def _sc_degree128(dst2):
    @functools.partial(
        pl.kernel,
        out_type=jax.ShapeDtypeStruct((NC, NPAD, D), jnp.float32),
        mesh=_mesh(),
        scratch_types=[
            pltpu.VMEM((ECH, 128), jnp.int32),
            pltpu.VMEM((128, D), jnp.float32),
            pltpu.SemaphoreType.DMA,
            pltpu.VMEM_SHARED((NPAD, D), jnp.float32),
        ],
    )
    def deg_kernel(dst_hbm, out_hbm, dst_v, ones_v, ssem, acc):
        cid = lax.axis_index("c")
        sid = lax.axis_index("s")
        wid = sid * NC + cid

        def fill(i, _):
            for k in range(D // 16):
                ones_v[i, pl.ds(k * 16, 16)] = jnp.zeros((16,), jnp.float32)
            return 0
        lax.fori_loop(0, 128, fill, 0)
        for k in range(RPT // 128):
            pltpu.sync_copy(ones_v, acc.at[pl.ds(sid * RPT + k * 128, 128)])

        def fill1(i, _):
            for k in range(D // 16):
                ones_v[i, pl.ds(k * 16, 16)] = jnp.full((16,), 1.0, jnp.float32)
            return 0
        lax.fori_loop(0, 128, fill1, 0)
        plsc.subcore_barrier()

        pltpu.sync_copy(dst_hbm.at[pl.ds(wid * ECH, ECH)], dst_v)

        def start_scatter(j):
            pltpu.async_copy(ones_v, acc.at[dst_v.at[j]], ssem, add=True)

        def wait_scatter(j):
            pltpu.make_async_copy(ones_v, acc.at[dst_v.at[j]], ssem).wait()

        def steps(jj, _):
            for k in range(4):
                j = 4 * jj + k
                start_scatter(j)
            for k in range(4):
                j = 4 * jj + k
                wait_scatter(j)
            return 0
        lax.fori_loop(0, ECH // 4, steps, 0)

        plsc.subcore_barrier()
        for k in range(RPT // 128):
            r0 = sid * RPT + k * 128
            pltpu.sync_copy(acc.at[pl.ds(r0, 128)], out_hbm.at[cid, pl.ds(r0, 128)])

    return deg_kernel(dst2)


def _sc_gather_pairs(t1, t2, els, eld):
    ept = SCH * 128

    @functools.partial(
        pl.kernel,
        out_type=[
            jax.ShapeDtypeStruct((ELPAD, D), jnp.float32),
            jax.ShapeDtypeStruct((ELPAD, D), jnp.float32),
        ],
        mesh=_mesh(),
        scratch_types=[
            pltpu.VMEM((ept,), jnp.int32),
            pltpu.VMEM((ept,), jnp.int32),
            pltpu.VMEM((128, D), jnp.float32),
            pltpu.VMEM((128, D), jnp.float32),
            pltpu.VMEM((128, D), jnp.float32),
            pltpu.VMEM((128, D), jnp.float32),
            pltpu.SemaphoreType.DMA,
            pltpu.SemaphoreType.DMA,
        ],
    )
    def gather_kernel(t1_hbm, t2_hbm, els_hbm, eld_hbm, outa_hbm, outb_hbm,
                      els_v, eld_v, ra0, ra1, rb0, rb1, sem_a, sem_b):
        rows_a = (ra0, ra1)
        rows_b = (rb0, rb1)
        cid = lax.axis_index("c")
        sid = lax.axis_index("s")
        wid = sid * NC + cid
        base = wid * ept

        pltpu.sync_copy(els_hbm.at[pl.ds(base, ept)], els_v)
        pltpu.sync_copy(eld_hbm.at[pl.ds(base, ept)], eld_v)

        def sg(t_hbm, idx_v, rows, j, k):
            pltpu.async_copy(
                t_hbm.at[idx_v.at[pl.ds(j * 128, 128)]], rows[k], sem_a)

        def wg(t_hbm, idx_v, rows, j, k):
            pltpu.make_async_copy(
                t_hbm.at[idx_v.at[pl.ds(j * 128, 128)]], rows[k], sem_a).wait()

        def sw(rows, out_hbm, j, k):
            pltpu.async_copy(rows[k], out_hbm.at[pl.ds(base + j * 128, 128)],
                             sem_b)

        def ww(rows, out_hbm, j, k):
            pltpu.make_async_copy(rows[k],
                                  out_hbm.at[pl.ds(base + j * 128, 128)],
                                  sem_b).wait()

        def step(j, k, first, last):
            for idx_v, rows, t_hbm, out_hbm in (
                (els_v, rows_a, t1_hbm, outa_hbm),
                (eld_v, rows_b, t2_hbm, outb_hbm),
            ):
                if not first:
                    ww(rows, out_hbm, j - 1, k)
                if not last:
                    sg(t_hbm, idx_v, rows, j + 1, k)
                wg(t_hbm, idx_v, rows, j, (k + 1) % 2)
                sw(rows, out_hbm, j, (k + 1) % 2)

        sg(t1_hbm, els_v, rows_a, 0, 0)
        sg(t2_hbm, eld_v, rows_b, 0, 0)
        step(0, 1, True, False)

        def steps(jj, _):
            for k in range(2):
                step(2 * jj + 1 + k, k, False, False)
            return 0
        lax.fori_loop(0, (SCH - 3) // 2, steps, 0)

        step(SCH - 2, 0, False, False)
        step(SCH - 1, 1, False, True)
        ww(rows_a, outa_hbm, SCH - 1, 0)
        ww(rows_b, outb_hbm, SCH - 1, 0)

    return gather_kernel(t1, t2, els, eld)


def _tc_score(pa, pb):
    R = 2048

    def body(a_ref, b_ref, out_ref):
        out_ref[...] = jnp.sum(a_ref[...] * b_ref[...], axis=1, keepdims=True)

    return pl.pallas_call(
        body,
        grid=(ELPAD // R,),
        in_specs=[
            pl.BlockSpec((R, D), lambda i: (i, 0)),
            pl.BlockSpec((R, D), lambda i: (i, 0)),
        ],
        out_specs=pl.BlockSpec((R, 1), lambda i: (i, 0)),
        out_shape=jax.ShapeDtypeStruct((ELPAD, 1), jnp.float32),
    )(pa, pb)


def _dinv_from(degp):
    d = degp[0, :, 0:1] + degp[1, :, 0:1] + 1.0
    return lax.rsqrt(d)


def _tc_pre(x, w1t, b1, w2t, b2, wc1t, degp):
    R = 1024

    def body(x_ref, w1_ref, b1_ref, w2_ref, b2_ref, wc1_ref, degp_ref, g1_ref):
        xv = x_ref[...]
        h = jnp.dot(xv, w1_ref[...], preferred_element_type=jnp.float32) + b1_ref[...]
        h = jnp.maximum(h, 0.01 * h)
        h = jnp.dot(h, w2_ref[...], preferred_element_type=jnp.float32) + b2_ref[...]
        h = jnp.maximum(h, 0.01 * h)
        hw = jnp.dot(h, wc1_ref[...], preferred_element_type=jnp.float32)
        g1_ref[...] = hw * _dinv_from(degp_ref[...])

    return pl.pallas_call(
        body,
        grid=(NPAD // R,),
        in_specs=[
            pl.BlockSpec((R, D), lambda i: (i, 0)),
            pl.BlockSpec((D, 256), lambda i: (0, 0)),
            pl.BlockSpec((1, 256), lambda i: (0, 0)),
            pl.BlockSpec((256, D), lambda i: (0, 0)),
            pl.BlockSpec((1, D), lambda i: (0, 0)),
            pl.BlockSpec((D, D), lambda i: (0, 0)),
            pl.BlockSpec((NC, R, 16), lambda i: (0, i, 0)),
        ],
        out_specs=pl.BlockSpec((R, D), lambda i: (i, 0)),
        out_shape=jax.ShapeDtypeStruct((NPAD, D), jnp.float32),
    )(x, w1t, b1, w2t, b2, wc1t, degp)


def _tc_mid(p, g1, degp, bc, wnt):
    R = 1024

    def body(p_ref, g_ref, degp_ref, bc_ref, w_ref, emb_ref, gn_ref):
        pv = p_ref[...]
        dinv = _dinv_from(degp_ref[...])
        z = (pv[0] + pv[1] + g_ref[...]) * dinv + bc_ref[...]
        emb = jnp.maximum(z, 0.01 * z)
        emb_ref[...] = emb
        gn_ref[...] = jnp.dot(emb, w_ref[...], preferred_element_type=jnp.float32) * dinv

    return pl.pallas_call(
        body,
        grid=(NPAD // R,),
        in_specs=[
            pl.BlockSpec((NC, R, D), lambda i: (0, i, 0)),
            pl.BlockSpec((R, D), lambda i: (i, 0)),
            pl.BlockSpec((NC, R, 16), lambda i: (0, i, 0)),
            pl.BlockSpec((1, D), lambda i: (0, 0)),
            pl.BlockSpec((D, D), lambda i: (0, 0)),
        ],
        out_specs=[
            pl.BlockSpec((R, D), lambda i: (i, 0)),
            pl.BlockSpec((R, D), lambda i: (i, 0)),
        ],
        out_shape=[
            jax.ShapeDtypeStruct((NPAD, D), jnp.float32),
            jax.ShapeDtypeStruct((NPAD, D), jnp.float32),
        ],
    )(p, g1, degp, bc, wnt)


def _tc_post(p, g2, degp, bc, wv):
    R = 1024

    def body(p_ref, g_ref, degp_ref, bc_ref, wv_ref, emb_ref, embw_ref):
        pv = p_ref[...]
        dinv = _dinv_from(degp_ref[...])
        z = (pv[0] + pv[1] + g_ref[...]) * dinv + bc_ref[...]
        emb = jnp.maximum(z, 0.01 * z)
        emb_ref[...] = emb
        embw_ref[...] = emb * wv_ref[...]

    return pl.pallas_call(
        body,
        grid=(NPAD // R,),
        in_specs=[
            pl.BlockSpec((NC, R, D), lambda i: (0, i, 0)),
            pl.BlockSpec((R, D), lambda i: (i, 0)),
            pl.BlockSpec((NC, R, 16), lambda i: (0, i, 0)),
            pl.BlockSpec((1, D), lambda i: (0, 0)),
            pl.BlockSpec((1, D), lambda i: (0, 0)),
        ],
        out_specs=[
            pl.BlockSpec((R, D), lambda i: (i, 0)),
            pl.BlockSpec((R, D), lambda i: (i, 0)),
        ],
        out_shape=[
            jax.ShapeDtypeStruct((NPAD, D), jnp.float32),
            jax.ShapeDtypeStruct((NPAD, D), jnp.float32),
        ],
    )(p, g2, degp, bc, wv)


def kernel(x, edge_index, edge_label_index,
           W_pre1, b_pre1, W_pre2, b_pre2,
           W_c1, b_c1, W_c2, b_c2, W_post, b_post):
    epad = N + jnp.arange(EPAD - E, dtype=jnp.int32) % (NPAD - N)
    src2 = jnp.concatenate([edge_index[0], epad]).reshape(EPAD // 128, 128)
    dst2 = jnp.concatenate([edge_index[1], epad]).reshape(EPAD // 128, 128)
    lpad = N + jnp.arange(ELPAD - EL, dtype=jnp.int32) % (NPAD - N)
    els2 = jnp.concatenate([edge_label_index[0], lpad])
    eld2 = jnp.concatenate([edge_label_index[1], lpad])
    x_pad = jnp.pad(x, ((0, NPAD - N), (0, 0)))

    degp = _sc_degree128(dst2)[:, :, :16]
    g1 = _tc_pre(x_pad, W_pre1.T, b_pre1[None], W_pre2.T, b_pre2[None],
                 W_c1.T, degp)
    p1 = _sc_conv(g1, src2, dst2)
    emb1, g2 = _tc_mid(p1, g1, degp, b_c1[None], W_c2.T)
    p2 = _sc_conv(g2, src2, dst2)
    wv = (W_post[0] + W_post[1])[None]
    emb2, emb2w = _tc_post(p2, g2, degp, b_c2[None], wv)
    pa, pb = _sc_gather_pairs(emb2w, emb2, els2, eld2)
    sco = _tc_score(pa, pb)
    scores = sco.reshape(ELPAD)[:EL] + (b_post[0] + b_post[1])
    return scores, emb1[:N], emb2[:N]


# speedup vs baseline: 20.6936x; 1.0438x over previous
import functools

import jax
import jax.numpy as jnp
from jax import lax
from jax.experimental import pallas as pl
from jax.experimental.pallas import tpu as pltpu
from jax.experimental.pallas import tpu_sc as plsc

N = 10000
D = 128
E = 320000
EL = 100000

NC = 2
NS = 16
NW = NC * NS

NPAD = 10240
EPAD = 327680
ELPAD = 102400
ECH = EPAD // NW // 128
SCH = ELPAD // NW // 128
RPT = NPAD // NS


def _mesh():
    return plsc.VectorSubcoreMesh(
        core_axis_name="c", subcore_axis_name="s", num_cores=NC, num_subcores=NS)


def _sc_conv(g, src2, dst2):
    H = ECH // 2

    @functools.partial(
        pl.kernel,
        out_type=jax.ShapeDtypeStruct((NC, NPAD, D), jnp.float32),
        mesh=_mesh(),
        scratch_types=[
            pltpu.VMEM((H, 128), jnp.int32),
            pltpu.VMEM((H, 128), jnp.int32),
            pltpu.VMEM((128, D), jnp.float32),
            pltpu.VMEM((128, D), jnp.float32),
            pltpu.SemaphoreType.DMA,
            pltpu.SemaphoreType.DMA,
            pltpu.VMEM_SHARED((NPAD, D), jnp.float32),
        ],
    )
    def conv_kernel(g_hbm, src_hbm, dst_hbm, out_hbm,
                    src_v, dst_v, rows0, rows1, gsem, ssem, acc):
        cid = lax.axis_index("c")
        sid = lax.axis_index("s")
        wid = sid * NC + cid
        bufs = (rows0, rows1)

        def start_gather(j, k):
            pltpu.async_copy(g_hbm.at[src_v.at[j]], bufs[k], gsem)

        def wait_gather(j, k):
            pltpu.make_async_copy(g_hbm.at[src_v.at[j]], bufs[k], gsem).wait()

        def start_scatter(j, k):
            pltpu.async_copy(bufs[k], acc.at[dst_v.at[j]], ssem, add=True)

        def wait_scatter(j, k):
            pltpu.make_async_copy(bufs[k], acc.at[dst_v.at[j]], ssem).wait()

        def zrow(i, _):
            for k in range(D // 16):
                rows0[i, pl.ds(k * 16, 16)] = jnp.zeros((16,), jnp.float32)
            return 0
        lax.fori_loop(0, 128, zrow, 0)
        for k in range(RPT // 128):
            pltpu.sync_copy(rows0, acc.at[pl.ds(sid * RPT + k * 128, 128)])
        plsc.subcore_barrier()

        for h in range(2):
            base = wid * ECH + h * H
            pltpu.sync_copy(src_hbm.at[pl.ds(base, H)], src_v)
            pltpu.sync_copy(dst_hbm.at[pl.ds(base, H)], dst_v)

            start_gather(0, 0)
            start_gather(1, 1)
            wait_gather(0, 0)
            start_scatter(0, 0)

            def steps(jj, _):
                for k in range(2):
                    j = 2 * jj + 1 + k
                    wait_scatter(j - 1, k)
                    start_gather(j + 1, k)
                    wait_gather(j, (k + 1) % 2)
                    start_scatter(j, (k + 1) % 2)
                return 0
            lax.fori_loop(0, (H - 2) // 2, steps, 0)

            wait_scatter(H - 2, H % 2)
            wait_gather(H - 1, (H - 1) % 2)
            start_scatter(H - 1, (H - 1) % 2)
            wait_scatter(H - 1, (H - 1) % 2)

        plsc.subcore_barrier()
        for k in range(RPT // 128):
            r0 = sid * RPT + k * 128
            pltpu.sync_copy(acc.at[pl.ds(r0, 128)], out_hbm.at[cid, pl.ds(r0, 128)])

    return conv_kernel(g, src2, dst2)


def _sc_degree128(dst2):
    @functools.partial(
        pl.kernel,
        out_type=jax.ShapeDtypeStruct((NC, NPAD, D), jnp.float32),
        mesh=_mesh(),
        scratch_types=[
            pltpu.VMEM((ECH, 128), jnp.int32),
            pltpu.VMEM((128, D), jnp.float32),
            pltpu.SemaphoreType.DMA,
            pltpu.VMEM_SHARED((NPAD, D), jnp.float32),
        ],
    )
    def deg_kernel(dst_hbm, out_hbm, dst_v, ones_v, ssem, acc):
        cid = lax.axis_index("c")
        sid = lax.axis_index("s")
        wid = sid * NC + cid

        def fill(i, _):
            for k in range(D // 16):
                ones_v[i, pl.ds(k * 16, 16)] = jnp.zeros((16,), jnp.float32)
            return 0
        lax.fori_loop(0, 128, fill, 0)
        for k in range(RPT // 128):
            pltpu.sync_copy(ones_v, acc.at[pl.ds(sid * RPT + k * 128, 128)])

        def fill1(i, _):
            for k in range(D // 16):
                ones_v[i, pl.ds(k * 16, 16)] = jnp.full((16,), 1.0, jnp.float32)
            return 0
        lax.fori_loop(0, 128, fill1, 0)
        plsc.subcore_barrier()

        pltpu.sync_copy(dst_hbm.at[pl.ds(wid * ECH, ECH)], dst_v)

        def start_scatter(j):
            pltpu.async_copy(ones_v, acc.at[dst_v.at[j]], ssem, add=True)

        def wait_scatter(j):
            pltpu.make_async_copy(ones_v, acc.at[dst_v.at[j]], ssem).wait()

        def steps(jj, _):
            for k in range(4):
                j = 4 * jj + k
                start_scatter(j)
            for k in range(4):
                j = 4 * jj + k
                wait_scatter(j)
            return 0
        lax.fori_loop(0, ECH // 4, steps, 0)

        plsc.subcore_barrier()
        for k in range(RPT // 128):
            r0 = sid * RPT + k * 128
            pltpu.sync_copy(acc.at[pl.ds(r0, 128)], out_hbm.at[cid, pl.ds(r0, 128)])

    return deg_kernel(dst2)


def _sc_gather_pairs(t1, t2, els, eld):
    ept = SCH * 128

    @functools.partial(
        pl.kernel,
        out_type=[
            jax.ShapeDtypeStruct((ELPAD, D), jnp.float32),
            jax.ShapeDtypeStruct((ELPAD, D), jnp.float32),
        ],
        mesh=_mesh(),
        scratch_types=[
            pltpu.VMEM((ept,), jnp.int32),
            pltpu.VMEM((ept,), jnp.int32),
            pltpu.VMEM((128, D), jnp.float32),
            pltpu.VMEM((128, D), jnp.float32),
            pltpu.VMEM((128, D), jnp.float32),
            pltpu.VMEM((128, D), jnp.float32),
            pltpu.SemaphoreType.DMA,
            pltpu.SemaphoreType.DMA,
        ],
    )
    def gather_kernel(t1_hbm, t2_hbm, els_hbm, eld_hbm, outa_hbm, outb_hbm,
                      els_v, eld_v, ra0, ra1, rb0, rb1, sem_a, sem_b):
        rows_a = (ra0, ra1)
        rows_b = (rb0, rb1)
        cid = lax.axis_index("c")
        sid = lax.axis_index("s")
        wid = sid * NC + cid
        base = wid * ept

        pltpu.sync_copy(els_hbm.at[pl.ds(base, ept)], els_v)
        pltpu.sync_copy(eld_hbm.at[pl.ds(base, ept)], eld_v)

        def sg(t_hbm, idx_v, rows, j, k):
            pltpu.async_copy(
                t_hbm.at[idx_v.at[pl.ds(j * 128, 128)]], rows[k], sem_a)

        def wg(t_hbm, idx_v, rows, j, k):
            pltpu.make_async_copy(
                t_hbm.at[idx_v.at[pl.ds(j * 128, 128)]], rows[k], sem_a).wait()

        def sw(rows, out_hbm, j, k):
            pltpu.async_copy(rows[k], out_hbm.at[pl.ds(base + j * 128, 128)],
                             sem_b)

        def ww(rows, out_hbm, j, k):
            pltpu.make_async_copy(rows[k],
                                  out_hbm.at[pl.ds(base + j * 128, 128)],
                                  sem_b).wait()

        def step(j, k, first, last):
            for idx_v, rows, t_hbm, out_hbm in (
                (els_v, rows_a, t1_hbm, outa_hbm),
                (eld_v, rows_b, t2_hbm, outb_hbm),
            ):
                if not first:
                    ww(rows, out_hbm, j - 1, k)
                if not last:
                    sg(t_hbm, idx_v, rows, j + 1, k)
                wg(t_hbm, idx_v, rows, j, (k + 1) % 2)
                sw(rows, out_hbm, j, (k + 1) % 2)

        sg(t1_hbm, els_v, rows_a, 0, 0)
        sg(t2_hbm, eld_v, rows_b, 0, 0)
        step(0, 1, True, False)

        def steps(jj, _):
            for k in range(2):
                step(2 * jj + 1 + k, k, False, False)
            return 0
        lax.fori_loop(0, (SCH - 3) // 2, steps, 0)

        step(SCH - 2, 0, False, False)
        step(SCH - 1, 1, False, True)
        ww(rows_a, outa_hbm, SCH - 1, 0)
        ww(rows_b, outb_hbm, SCH - 1, 0)

    return gather_kernel(t1, t2, els, eld)


def _tc_score(pa, pb, bsum):
    R = 2048

    def body(a_ref, b_ref, bs_ref, out_ref):
        s = jnp.sum(a_ref[...] * b_ref[...], axis=1)
        out_ref[...] = s + bs_ref[0, 0]

    return pl.pallas_call(
        body,
        grid=(pl.cdiv(EL, R),),
        in_specs=[
            pl.BlockSpec((R, D), lambda i: (i, 0)),
            pl.BlockSpec((R, D), lambda i: (i, 0)),
            pl.BlockSpec((1, 1), lambda i: (0, 0)),
        ],
        out_specs=pl.BlockSpec((R,), lambda i: (i,)),
        out_shape=jax.ShapeDtypeStruct((EL,), jnp.float32),
    )(pa, pb, bsum)


def _dinv_from(degp):
    d = degp[0, :, 0:1] + degp[1, :, 0:1] + 1.0
    return lax.rsqrt(d)


def _tc_mlp(x, w1t, b1, w2t, b2, wc1t):
    R = 1024

    def body(x_ref, w1_ref, b1_ref, w2_ref, b2_ref, wc1_ref, hw_ref):
        xv = x_ref[...]
        h = jnp.dot(xv, w1_ref[...], preferred_element_type=jnp.float32) + b1_ref[...]
        h = jnp.maximum(h, 0.01 * h)
        h = jnp.dot(h, w2_ref[...], preferred_element_type=jnp.float32) + b2_ref[...]
        h = jnp.maximum(h, 0.01 * h)
        hw_ref[...] = jnp.dot(h, wc1_ref[...], preferred_element_type=jnp.float32)

    return pl.pallas_call(
        body,
        grid=(NPAD // R,),
        in_specs=[
            pl.BlockSpec((R, D), lambda i: (i, 0)),
            pl.BlockSpec((D, 256), lambda i: (0, 0)),
            pl.BlockSpec((1, 256), lambda i: (0, 0)),
            pl.BlockSpec((256, D), lambda i: (0, 0)),
            pl.BlockSpec((1, D), lambda i: (0, 0)),
            pl.BlockSpec((D, D), lambda i: (0, 0)),
        ],
        out_specs=pl.BlockSpec((R, D), lambda i: (i, 0)),
        out_shape=jax.ShapeDtypeStruct((NPAD, D), jnp.float32),
    )(x, w1t, b1, w2t, b2, wc1t)


def _tc_scale(hw, degp):
    R = 1024

    def body(hw_ref, degp_ref, g_ref):
        g_ref[...] = hw_ref[...] * _dinv_from(degp_ref[...])

    return pl.pallas_call(
        body,
        grid=(NPAD // R,),
        in_specs=[
            pl.BlockSpec((R, D), lambda i: (i, 0)),
            pl.BlockSpec((NC, R, 16), lambda i: (0, i, 0)),
        ],
        out_specs=pl.BlockSpec((R, D), lambda i: (i, 0)),
        out_shape=jax.ShapeDtypeStruct((NPAD, D), jnp.float32),
    )(hw, degp)


def _tc_mid(p, g1, degp, bc, wnt):
    R = 1024

    def body(p_ref, g_ref, degp_ref, bc_ref, w_ref, emb_ref, gn_ref):
        pv = p_ref[...]
        dinv = _dinv_from(degp_ref[...])
        z = (pv[0] + pv[1] + g_ref[...]) * dinv + bc_ref[...]
        emb = jnp.maximum(z, 0.01 * z)
        emb_ref[...] = emb
        gn_ref[...] = jnp.dot(emb, w_ref[...], preferred_element_type=jnp.float32) * dinv

    return pl.pallas_call(
        body,
        grid=(NPAD // R,),
        in_specs=[
            pl.BlockSpec((NC, R, D), lambda i: (0, i, 0)),
            pl.BlockSpec((R, D), lambda i: (i, 0)),
            pl.BlockSpec((NC, R, 16), lambda i: (0, i, 0)),
            pl.BlockSpec((1, D), lambda i: (0, 0)),
            pl.BlockSpec((D, D), lambda i: (0, 0)),
        ],
        out_specs=[
            pl.BlockSpec((R, D), lambda i: (i, 0)),
            pl.BlockSpec((R, D), lambda i: (i, 0)),
        ],
        out_shape=[
            jax.ShapeDtypeStruct((N, D), jnp.float32),
            jax.ShapeDtypeStruct((NPAD, D), jnp.float32),
        ],
    )(p, g1, degp, bc, wnt)


def _tc_post(p, g2, degp, bc, wv):
    R = 1024

    def body(p_ref, g_ref, degp_ref, bc_ref, wv_ref, emb_ref, embw_ref):
        pv = p_ref[...]
        dinv = _dinv_from(degp_ref[...])
        z = (pv[0] + pv[1] + g_ref[...]) * dinv + bc_ref[...]
        emb = jnp.maximum(z, 0.01 * z)
        emb_ref[...] = emb
        embw_ref[...] = emb * wv_ref[...]

    return pl.pallas_call(
        body,
        grid=(NPAD // R,),
        in_specs=[
            pl.BlockSpec((NC, R, D), lambda i: (0, i, 0)),
            pl.BlockSpec((R, D), lambda i: (i, 0)),
            pl.BlockSpec((NC, R, 16), lambda i: (0, i, 0)),
            pl.BlockSpec((1, D), lambda i: (0, 0)),
            pl.BlockSpec((1, D), lambda i: (0, 0)),
        ],
        out_specs=[
            pl.BlockSpec((R, D), lambda i: (i, 0)),
            pl.BlockSpec((R, D), lambda i: (i, 0)),
        ],
        out_shape=[
            jax.ShapeDtypeStruct((N, D), jnp.float32),
            jax.ShapeDtypeStruct((N, D), jnp.float32),
        ],
    )(p, g2, degp, bc, wv)


def kernel(x, edge_index, edge_label_index,
           W_pre1, b_pre1, W_pre2, b_pre2,
           W_c1, b_c1, W_c2, b_c2, W_post, b_post):
    epad = N + jnp.arange(EPAD - E, dtype=jnp.int32) % (NPAD - N)
    src2 = jnp.concatenate([edge_index[0], epad]).reshape(EPAD // 128, 128)
    dst2 = jnp.concatenate([edge_index[1], epad]).reshape(EPAD // 128, 128)
    lpad = jnp.arange(ELPAD - EL, dtype=jnp.int32) % N
    els2 = jnp.concatenate([edge_label_index[0], lpad])
    eld2 = jnp.concatenate([edge_label_index[1], lpad])
    x_pad = jnp.pad(x, ((0, NPAD - N), (0, 0)))

    degp = _sc_degree128(dst2)[:, :, :16]
    hw1 = _tc_mlp(x_pad, W_pre1.T, b_pre1[None], W_pre2.T, b_pre2[None],
                  W_c1.T)
    g1 = _tc_scale(hw1, degp)
    p1 = _sc_conv(g1, src2, dst2)
    emb1, g2 = _tc_mid(p1, g1, degp, b_c1[None], W_c2.T)
    p2 = _sc_conv(g2, src2, dst2)
    wv = (W_post[0] + W_post[1])[None]
    emb2, emb2w = _tc_post(p2, g2, degp, b_c2[None], wv)
    pa, pb = _sc_gather_pairs(emb2w, emb2, els2, eld2)
    scores = _tc_score(pa, pb, (b_post[0] + b_post[1]).reshape(1, 1))
    return scores, emb1, emb2


# speedup vs baseline: 20.9859x; 1.0141x over previous
import functools

import jax
import jax.numpy as jnp
from jax import lax
from jax.experimental import pallas as pl
from jax.experimental.pallas import tpu as pltpu
from jax.experimental.pallas import tpu_sc as plsc

N = 10000
D = 128
E = 320000
EL = 100000

NC = 2
NS = 16
NW = NC * NS

NPAD = 10240
EPAD = 327680
ELPAD = 102400
ECH = EPAD // NW // 128
SCH = ELPAD // NW // 128
RPT = NPAD // NS


def _mesh():
    return plsc.VectorSubcoreMesh(
        core_axis_name="c", subcore_axis_name="s", num_cores=NC, num_subcores=NS)


def _sc_conv(g, src2, dst2):
    H = ECH // 2

    @functools.partial(
        pl.kernel,
        out_type=jax.ShapeDtypeStruct((NC, NPAD, D), jnp.float32),
        mesh=_mesh(),
        scratch_types=[
            pltpu.VMEM((H, 128), jnp.int32),
            pltpu.VMEM((H, 128), jnp.int32),
            pltpu.VMEM((128, D), jnp.float32),
            pltpu.VMEM((128, D), jnp.float32),
            pltpu.SemaphoreType.DMA,
            pltpu.SemaphoreType.DMA,
            pltpu.VMEM_SHARED((NPAD, D), jnp.float32),
        ],
    )
    def conv_kernel(g_hbm, src_hbm, dst_hbm, out_hbm,
                    src_v, dst_v, rows0, rows1, gsem, ssem, acc):
        cid = lax.axis_index("c")
        sid = lax.axis_index("s")
        wid = sid * NC + cid
        bufs = (rows0, rows1)

        def start_gather(j, k):
            pltpu.async_copy(g_hbm.at[src_v.at[j]], bufs[k], gsem)

        def wait_gather(j, k):
            pltpu.make_async_copy(g_hbm.at[src_v.at[j]], bufs[k], gsem).wait()

        def start_scatter(j, k):
            pltpu.async_copy(bufs[k], acc.at[dst_v.at[j]], ssem, add=True)

        def wait_scatter(j, k):
            pltpu.make_async_copy(bufs[k], acc.at[dst_v.at[j]], ssem).wait()

        def zrow(i, _):
            for k in range(D // 16):
                rows0[i, pl.ds(k * 16, 16)] = jnp.zeros((16,), jnp.float32)
            return 0
        lax.fori_loop(0, 128, zrow, 0)
        for k in range(RPT // 128):
            pltpu.sync_copy(rows0, acc.at[pl.ds(sid * RPT + k * 128, 128)])
        plsc.subcore_barrier()

        for h in range(2):
            base = wid * ECH + h * H
            pltpu.sync_copy(src_hbm.at[pl.ds(base, H)], src_v)
            pltpu.sync_copy(dst_hbm.at[pl.ds(base, H)], dst_v)

            start_gather(0, 0)
            start_gather(1, 1)
            wait_gather(0, 0)
            start_scatter(0, 0)

            def steps(jj, _):
                for k in range(2):
                    j = 2 * jj + 1 + k
                    wait_scatter(j - 1, k)
                    start_gather(j + 1, k)
                    wait_gather(j, (k + 1) % 2)
                    start_scatter(j, (k + 1) % 2)
                return 0
            lax.fori_loop(0, (H - 2) // 2, steps, 0)

            wait_scatter(H - 2, H % 2)
            wait_gather(H - 1, (H - 1) % 2)
            start_scatter(H - 1, (H - 1) % 2)
            wait_scatter(H - 1, (H - 1) % 2)

        plsc.subcore_barrier()
        for k in range(RPT // 128):
            r0 = sid * RPT + k * 128
            pltpu.sync_copy(acc.at[pl.ds(r0, 128)], out_hbm.at[cid, pl.ds(r0, 128)])

    return conv_kernel(g, src2, dst2)


def _sc_degree128(dst2):
    @functools.partial(
        pl.kernel,
        out_type=jax.ShapeDtypeStruct((NC, NPAD, D), jnp.float32),
        mesh=_mesh(),
        scratch_types=[
            pltpu.VMEM((ECH, 128), jnp.int32),
            pltpu.VMEM((128, D), jnp.float32),
            pltpu.SemaphoreType.DMA,
            pltpu.VMEM_SHARED((NPAD, D), jnp.float32),
        ],
    )
    def deg_kernel(dst_hbm, out_hbm, dst_v, ones_v, ssem, acc):
        cid = lax.axis_index("c")
        sid = lax.axis_index("s")
        wid = sid * NC + cid

        def fill(i, _):
            for k in range(D // 16):
                ones_v[i, pl.ds(k * 16, 16)] = jnp.zeros((16,), jnp.float32)
            return 0
        lax.fori_loop(0, 128, fill, 0)
        for k in range(RPT // 128):
            pltpu.sync_copy(ones_v, acc.at[pl.ds(sid * RPT + k * 128, 128)])

        def fill1(i, _):
            for k in range(D // 16):
                ones_v[i, pl.ds(k * 16, 16)] = jnp.full((16,), 1.0, jnp.float32)
            return 0
        lax.fori_loop(0, 128, fill1, 0)
        plsc.subcore_barrier()

        pltpu.sync_copy(dst_hbm.at[pl.ds(wid * ECH, ECH)], dst_v)

        def start_scatter(j):
            pltpu.async_copy(ones_v, acc.at[dst_v.at[j]], ssem, add=True)

        def wait_scatter(j):
            pltpu.make_async_copy(ones_v, acc.at[dst_v.at[j]], ssem).wait()

        def steps(jj, _):
            for k in range(4):
                j = 4 * jj + k
                start_scatter(j)
            for k in range(4):
                j = 4 * jj + k
                wait_scatter(j)
            return 0
        lax.fori_loop(0, ECH // 4, steps, 0)

        plsc.subcore_barrier()
        for k in range(RPT // 128):
            r0 = sid * RPT + k * 128
            pltpu.sync_copy(acc.at[pl.ds(r0, 128)], out_hbm.at[cid, pl.ds(r0, 128)])

    return deg_kernel(dst2)


def _sc_gather_pairs(t1, t2, els, eld, sch):
    ept = sch * 128
    elp = NW * ept

    @functools.partial(
        pl.kernel,
        out_type=[
            jax.ShapeDtypeStruct((elp, D), jnp.float32),
            jax.ShapeDtypeStruct((elp, D), jnp.float32),
        ],
        mesh=_mesh(),
        scratch_types=[
            pltpu.VMEM((ept,), jnp.int32),
            pltpu.VMEM((ept,), jnp.int32),
            pltpu.VMEM((128, D), jnp.float32),
            pltpu.VMEM((128, D), jnp.float32),
            pltpu.VMEM((128, D), jnp.float32),
            pltpu.VMEM((128, D), jnp.float32),
            pltpu.SemaphoreType.DMA,
            pltpu.SemaphoreType.DMA,
        ],
    )
    def gather_kernel(t1_hbm, t2_hbm, els_hbm, eld_hbm, outa_hbm, outb_hbm,
                      els_v, eld_v, ra0, ra1, rb0, rb1, sem_a, sem_b):
        rows_a = (ra0, ra1)
        rows_b = (rb0, rb1)
        cid = lax.axis_index("c")
        sid = lax.axis_index("s")
        wid = sid * NC + cid
        base = wid * ept

        pltpu.sync_copy(els_hbm.at[pl.ds(base, ept)], els_v)
        pltpu.sync_copy(eld_hbm.at[pl.ds(base, ept)], eld_v)

        def sg(t_hbm, idx_v, rows, j, k):
            pltpu.async_copy(
                t_hbm.at[idx_v.at[pl.ds(j * 128, 128)]], rows[k], sem_a)

        def wg(t_hbm, idx_v, rows, j, k):
            pltpu.make_async_copy(
                t_hbm.at[idx_v.at[pl.ds(j * 128, 128)]], rows[k], sem_a).wait()

        def sw(rows, out_hbm, j, k):
            pltpu.async_copy(rows[k], out_hbm.at[pl.ds(base + j * 128, 128)],
                             sem_b)

        def ww(rows, out_hbm, j, k):
            pltpu.make_async_copy(rows[k],
                                  out_hbm.at[pl.ds(base + j * 128, 128)],
                                  sem_b).wait()

        def step(j, k, first, last):
            for idx_v, rows, t_hbm, out_hbm in (
                (els_v, rows_a, t1_hbm, outa_hbm),
                (eld_v, rows_b, t2_hbm, outb_hbm),
            ):
                if not first:
                    ww(rows, out_hbm, j - 1, k)
                if not last:
                    sg(t_hbm, idx_v, rows, j + 1, k)
                wg(t_hbm, idx_v, rows, j, (k + 1) % 2)
                sw(rows, out_hbm, j, (k + 1) % 2)

        sg(t1_hbm, els_v, rows_a, 0, 0)
        sg(t2_hbm, eld_v, rows_b, 0, 0)
        step(0, 1, True, False)

        def steps(jj, _):
            for k in range(2):
                step(2 * jj + 1 + k, k, False, False)
            return 0
        lax.fori_loop(0, (sch - 3) // 2, steps, 0)

        step(sch - 2, 0, False, False)
        step(sch - 1, 1, False, True)
        ww(rows_a, outa_hbm, sch - 1, 0)
        ww(rows_b, outb_hbm, sch - 1, 0)

    return gather_kernel(t1, t2, els, eld)


def _tc_score(pa, pb, bsum, n_out):
    R = 2048

    def body(a_ref, b_ref, bs_ref, out_ref):
        s = jnp.sum(a_ref[...] * b_ref[...], axis=1)
        out_ref[...] = s + bs_ref[0, 0]

    return pl.pallas_call(
        body,
        grid=(pl.cdiv(n_out, R),),
        in_specs=[
            pl.BlockSpec((R, D), lambda i: (i, 0)),
            pl.BlockSpec((R, D), lambda i: (i, 0)),
            pl.BlockSpec((1, 1), lambda i: (0, 0)),
        ],
        out_specs=pl.BlockSpec((R,), lambda i: (i,)),
        out_shape=jax.ShapeDtypeStruct((n_out,), jnp.float32),
    )(pa, pb, bsum)


def _dinv_from(degp):
    d = degp[0, :, 0:1] + degp[1, :, 0:1] + 1.0
    return lax.rsqrt(d)


def _tc_mlp(x, w1t, b1, w2t, b2, wc1t):
    R = 1024

    def body(x_ref, w1_ref, b1_ref, w2_ref, b2_ref, wc1_ref, hw_ref):
        xv = x_ref[...]
        h = jnp.dot(xv, w1_ref[...], preferred_element_type=jnp.float32) + b1_ref[...]
        h = jnp.maximum(h, 0.01 * h)
        h = jnp.dot(h, w2_ref[...], preferred_element_type=jnp.float32) + b2_ref[...]
        h = jnp.maximum(h, 0.01 * h)
        hw_ref[...] = jnp.dot(h, wc1_ref[...], preferred_element_type=jnp.float32)

    return pl.pallas_call(
        body,
        grid=(NPAD // R,),
        in_specs=[
            pl.BlockSpec((R, D), lambda i: (i, 0)),
            pl.BlockSpec((D, 256), lambda i: (0, 0)),
            pl.BlockSpec((1, 256), lambda i: (0, 0)),
            pl.BlockSpec((256, D), lambda i: (0, 0)),
            pl.BlockSpec((1, D), lambda i: (0, 0)),
            pl.BlockSpec((D, D), lambda i: (0, 0)),
        ],
        out_specs=pl.BlockSpec((R, D), lambda i: (i, 0)),
        out_shape=jax.ShapeDtypeStruct((NPAD, D), jnp.float32),
    )(x, w1t, b1, w2t, b2, wc1t)


def _tc_scale(hw, degp):
    R = 1024

    def body(hw_ref, degp_ref, g_ref):
        g_ref[...] = hw_ref[...] * _dinv_from(degp_ref[...])

    return pl.pallas_call(
        body,
        grid=(NPAD // R,),
        in_specs=[
            pl.BlockSpec((R, D), lambda i: (i, 0)),
            pl.BlockSpec((NC, R, 16), lambda i: (0, i, 0)),
        ],
        out_specs=pl.BlockSpec((R, D), lambda i: (i, 0)),
        out_shape=jax.ShapeDtypeStruct((NPAD, D), jnp.float32),
    )(hw, degp)


def _tc_mid(p, g1, degp, bc, wnt):
    R = 1024

    def body(p_ref, g_ref, degp_ref, bc_ref, w_ref, emb_ref, gn_ref):
        pv = p_ref[...]
        dinv = _dinv_from(degp_ref[...])
        z = (pv[0] + pv[1] + g_ref[...]) * dinv + bc_ref[...]
        emb = jnp.maximum(z, 0.01 * z)
        emb_ref[...] = emb
        gn_ref[...] = jnp.dot(emb, w_ref[...], preferred_element_type=jnp.float32) * dinv

    return pl.pallas_call(
        body,
        grid=(NPAD // R,),
        in_specs=[
            pl.BlockSpec((NC, R, D), lambda i: (0, i, 0)),
            pl.BlockSpec((R, D), lambda i: (i, 0)),
            pl.BlockSpec((NC, R, 16), lambda i: (0, i, 0)),
            pl.BlockSpec((1, D), lambda i: (0, 0)),
            pl.BlockSpec((D, D), lambda i: (0, 0)),
        ],
        out_specs=[
            pl.BlockSpec((R, D), lambda i: (i, 0)),
            pl.BlockSpec((R, D), lambda i: (i, 0)),
        ],
        out_shape=[
            jax.ShapeDtypeStruct((N, D), jnp.float32),
            jax.ShapeDtypeStruct((NPAD, D), jnp.float32),
        ],
    )(p, g1, degp, bc, wnt)


def _tc_post(p, g2, degp, bc, wv):
    R = 1024

    def body(p_ref, g_ref, degp_ref, bc_ref, wv_ref, emb_ref, embw_ref):
        pv = p_ref[...]
        dinv = _dinv_from(degp_ref[...])
        z = (pv[0] + pv[1] + g_ref[...]) * dinv + bc_ref[...]
        emb = jnp.maximum(z, 0.01 * z)
        emb_ref[...] = emb
        embw_ref[...] = emb * wv_ref[...]

    return pl.pallas_call(
        body,
        grid=(NPAD // R,),
        in_specs=[
            pl.BlockSpec((NC, R, D), lambda i: (0, i, 0)),
            pl.BlockSpec((R, D), lambda i: (i, 0)),
            pl.BlockSpec((NC, R, 16), lambda i: (0, i, 0)),
            pl.BlockSpec((1, D), lambda i: (0, 0)),
            pl.BlockSpec((1, D), lambda i: (0, 0)),
        ],
        out_specs=[
            pl.BlockSpec((R, D), lambda i: (i, 0)),
            pl.BlockSpec((R, D), lambda i: (i, 0)),
        ],
        out_shape=[
            jax.ShapeDtypeStruct((N, D), jnp.float32),
            jax.ShapeDtypeStruct((N, D), jnp.float32),
        ],
    )(p, g2, degp, bc, wv)


def kernel(x, edge_index, edge_label_index,
           W_pre1, b_pre1, W_pre2, b_pre2,
           W_c1, b_c1, W_c2, b_c2, W_post, b_post):
    epad = N + jnp.arange(EPAD - E, dtype=jnp.int32) % (NPAD - N)
    src2 = jnp.concatenate([edge_index[0], epad]).reshape(EPAD // 128, 128)
    dst2 = jnp.concatenate([edge_index[1], epad]).reshape(EPAD // 128, 128)
    ELH = EL // 2
    ELHP = NW * 13 * 128
    lpad = jnp.arange(ELHP - ELH, dtype=jnp.int32) % N
    els_a = jnp.concatenate([edge_label_index[0, :ELH], lpad])
    eld_a = jnp.concatenate([edge_label_index[1, :ELH], lpad])
    els_b = jnp.concatenate([edge_label_index[0, ELH:], lpad])
    eld_b = jnp.concatenate([edge_label_index[1, ELH:], lpad])
    x_pad = jnp.pad(x, ((0, NPAD - N), (0, 0)))

    degp = _sc_degree128(dst2)[:, :, :16]
    hw1 = _tc_mlp(x_pad, W_pre1.T, b_pre1[None], W_pre2.T, b_pre2[None],
                  W_c1.T)
    g1 = _tc_scale(hw1, degp)
    p1 = _sc_conv(g1, src2, dst2)
    emb1, g2 = _tc_mid(p1, g1, degp, b_c1[None], W_c2.T)
    p2 = _sc_conv(g2, src2, dst2)
    wv = (W_post[0] + W_post[1])[None]
    emb2, emb2w = _tc_post(p2, g2, degp, b_c2[None], wv)
    bsum = (b_post[0] + b_post[1]).reshape(1, 1)
    pa1, pb1 = _sc_gather_pairs(emb2w, emb2, els_a, eld_a, 13)
    pa2, pb2 = _sc_gather_pairs(emb2w, emb2, els_b, eld_b, 13)
    s1 = _tc_score(pa1, pb1, bsum, ELH)
    s2 = _tc_score(pa2, pb2, bsum, ELH)
    scores = jnp.concatenate([s1, s2])
    return scores, emb1, emb2
